# baseline (speedup 1.0000x reference)
import jax
import jax.numpy as jnp
from jax import lax
from jax.experimental import pallas as pl
from jax.experimental.pallas import tpu as pltpu
from jax.experimental.pallas import tpu_sc as plsc

N = 10
M = 64
KPTS = 16
LANES = 16
GROUPS = M // LANES
TASKS = N * GROUPS
NW = 4
GWORDS = KPTS * 3 * LANES
KTOT = GROUPS * GWORDS
XOFF = KTOT
XCOPY = 32


def _sc_body(packed_hbm, out_hbm, xv, kv, ov, sem0):
    w = lax.axis_index("s")
    g = w % GROUPS

    def compute(i):
        xr = xv[pl.ds(i * 3, LANES)]
        x0 = jnp.full((LANES,), xr[0], jnp.float32)
        x1 = jnp.full((LANES,), xr[1], jnp.float32)
        x2 = jnp.full((LANES,), xr[2], jnp.float32)
        acc = None
        for l in range(KPTS):
            d0 = x0 - kv[pl.ds((l * 3 + 0) * LANES, LANES)]
            d1 = x1 - kv[pl.ds((l * 3 + 1) * LANES, LANES)]
            d2 = x2 - kv[pl.ds((l * 3 + 2) * LANES, LANES)]
            e = jnp.exp(-(d0 * d0 + d1 * d1 + d2 * d2))
            acc = e if acc is None else acc + e
        ov[...] = acc * (1.0 / 128.0)
        pltpu.sync_copy(ov, out_hbm.at[i, pl.ds(g * LANES, LANES)])

    cx = pltpu.make_async_copy(packed_hbm.at[pl.ds(XOFF, XCOPY)],
                               xv.at[pl.ds(0, XCOPY)], sem0)
    cx.start()
    ck = pltpu.make_async_copy(packed_hbm.at[pl.ds(g * GWORDS, GWORDS)],
                               kv, sem0)
    ck.start()
    cx.wait()
    ck.wait()
    for j in range(N):
        compute(j)


@jax.jit
def _run(normal, learnable_kernel):
    kt = (learnable_kernel.reshape(GROUPS, LANES, KPTS, 3)
          .transpose(0, 2, 3, 1)
          .reshape(KTOT))
    packed = jnp.concatenate([kt, normal[:N].reshape(N * 3),
                              jnp.zeros(2, jnp.float32)])
    sc_call = pl.kernel(
        _sc_body,
        out_type=jax.ShapeDtypeStruct((N, M), jnp.float32),
        mesh=plsc.VectorSubcoreMesh(core_axis_name="c", subcore_axis_name="s",
                                    num_cores=1, num_subcores=4),
        scratch_types=[
            pltpu.VMEM((48,), jnp.float32),
            pltpu.VMEM((GWORDS,), jnp.float32),
            pltpu.VMEM((LANES,), jnp.float32),
            pltpu.SemaphoreType.DMA,
        ],
    )
    return sc_call(packed)


def kernel(normal, neighbour, learnable_kernel):
    del neighbour
    return _run(normal, learnable_kernel)


# speedup vs baseline: 1.1286x; 1.1286x over previous
import jax
import jax.numpy as jnp
from jax import lax
from jax.experimental import pallas as pl
from jax.experimental.pallas import tpu as pltpu
from jax.experimental.pallas import tpu_sc as plsc

N = 10
M = 64
KPTS = 16
LANES = 16
GROUPS = M // LANES
TASKS = N * GROUPS
NW = 16
GWORDS = KPTS * 3 * LANES
KTOT = GROUPS * GWORDS
XOFF = KTOT
XCOPY = 32


def _sc_body(packed_hbm, out_hbm, xv, kv, ov, sem0):
    w = lax.axis_index("s")
    g = w % GROUPS
    i0 = w // GROUPS

    def compute(i):
        xr = xv[pl.ds(i * 3, LANES)]
        x0 = jnp.full((LANES,), xr[0], jnp.float32)
        x1 = jnp.full((LANES,), xr[1], jnp.float32)
        x2 = jnp.full((LANES,), xr[2], jnp.float32)
        acc = None
        for l in range(KPTS):
            d0 = x0 - kv[pl.ds((l * 3 + 0) * LANES, LANES)]
            d1 = x1 - kv[pl.ds((l * 3 + 1) * LANES, LANES)]
            d2 = x2 - kv[pl.ds((l * 3 + 2) * LANES, LANES)]
            e = jnp.exp(-(d0 * d0 + d1 * d1 + d2 * d2))
            acc = e if acc is None else acc + e
        ov[...] = acc * (1.0 / 128.0)
        pltpu.sync_copy(ov, out_hbm.at[i, pl.ds(g * LANES, LANES)])

    cx = pltpu.make_async_copy(packed_hbm.at[pl.ds(XOFF, XCOPY)],
                               xv.at[pl.ds(0, XCOPY)], sem0)
    cx.start()
    ck = pltpu.make_async_copy(packed_hbm.at[pl.ds(g * GWORDS, GWORDS)],
                               kv, sem0)
    ck.start()
    cx.wait()
    ck.wait()
    compute(i0)
    compute(i0 + GROUPS)

    @pl.when(w < TASKS - 2 * NW)
    def _third():
        compute(i0 + 2 * GROUPS)


@jax.jit
def _run(normal, learnable_kernel):
    kt = (learnable_kernel.reshape(GROUPS, LANES, KPTS, 3)
          .transpose(0, 2, 3, 1)
          .reshape(KTOT))
    packed = jnp.concatenate([kt, normal[:N].reshape(N * 3),
                              jnp.zeros(2, jnp.float32)])
    sc_call = pl.kernel(
        _sc_body,
        out_type=jax.ShapeDtypeStruct((N, M), jnp.float32),
        mesh=plsc.VectorSubcoreMesh(core_axis_name="c", subcore_axis_name="s",
                                    num_cores=1),
        scratch_types=[
            pltpu.VMEM((48,), jnp.float32),
            pltpu.VMEM((GWORDS,), jnp.float32),
            pltpu.VMEM((LANES,), jnp.float32),
            pltpu.SemaphoreType.DMA,
        ],
    )
    return sc_call(packed)


def kernel(normal, neighbour, learnable_kernel):
    del neighbour
    return _run(normal, learnable_kernel)
